# baseline (speedup 1.0000x reference)
import functools
import jax
import jax.numpy as jnp
from jax import lax
from jax.experimental import pallas as pl
from jax.experimental.pallas import tpu as pltpu
from jax.experimental.pallas import tpu_sc as plsc


def kernel(x_ids, table):
    bsz, seq_len = x_ids.shape
    d = table.shape[1]
    bsz_sc = 1
    bsz_tc = bsz - bsz_sc

    info = plsc.get_sparse_core_info()
    NC, NS = info.num_cores, info.num_subcores
    NW = NC * NS
    rows_per_w = seq_len // NW
    C = 32
    n_chunks = rows_per_w // C

    mesh = plsc.VectorSubcoreMesh(core_axis_name="c", subcore_axis_name="s")

    @functools.partial(
        pl.kernel,
        mesh=mesh,
        out_type=jax.ShapeDtypeStruct((bsz_sc, seq_len, d), table.dtype),
        scratch_types=[
            pltpu.VMEM((C, d), table.dtype),
            pltpu.VMEM((C, d), table.dtype),
            pltpu.SemaphoreType.DMA,
            pltpu.SemaphoreType.DMA,
            pltpu.SemaphoreType.DMA,
        ],
    )
    def sc_copy(table_hbm, out_hbm, buf0, buf1, gsem, wsem0, wsem1):
        wid = lax.axis_index("s") * NC + lax.axis_index("c")
        base = wid * rows_per_w
        bufs = (buf0, buf1)
        wsems = (wsem0, wsem1)
        gathers = [None, None]
        scatters = [[], []]
        gathers[0] = pltpu.async_copy(table_hbm.at[pl.ds(base, C)], bufs[0], gsem)
        for i in range(n_chunks):
            k = i % 2
            gathers[k].wait()
            if i + 1 < n_chunks:
                nk = (i + 1) % 2
                for h in scatters[nk]:
                    h.wait()
                scatters[nk] = []
                gathers[nk] = pltpu.async_copy(
                    table_hbm.at[pl.ds(base + (i + 1) * C, C)], bufs[nk], gsem
                )
            start = base + i * C
            for b in range(bsz_sc):
                scatters[k].append(
                    pltpu.async_copy(bufs[k], out_hbm.at[b, pl.ds(start, C)], wsems[k])
                )
        for k in (0, 1):
            for h in scatters[k]:
                h.wait()

    sc_out = sc_copy(table)

    BT = 1024
    nb = seq_len // BT

    def tc_body(tab_ref, out_ref):
        out_ref[...] = jnp.broadcast_to(tab_ref[...][None], out_ref.shape)

    tc_out = pl.pallas_call(
        tc_body,
        grid=(nb,),
        in_specs=[pl.BlockSpec((BT, d), lambda j: (j, 0))],
        out_specs=pl.BlockSpec((bsz_tc, BT, d), lambda j: (0, j, 0)),
        out_shape=jax.ShapeDtypeStruct((bsz_tc, seq_len, d), table.dtype),
    )(table)

    return jnp.concatenate([tc_out, sc_out], axis=0)


# speedup vs baseline: 2.0505x; 2.0505x over previous
import functools
import jax
import jax.numpy as jnp
from jax import lax
from jax.experimental import pallas as pl
from jax.experimental.pallas import tpu as pltpu
from jax.experimental.pallas import tpu_sc as plsc


def kernel(x_ids, table):
    bsz, seq_len = x_ids.shape
    d = table.shape[1]

    info = plsc.get_sparse_core_info()
    NC, NS = info.num_cores, info.num_subcores
    NW = NC * NS
    rows_per_w = seq_len // NW
    C = 16
    NBUF = 4
    AHEAD = 2
    n_chunks = rows_per_w // C

    mesh = plsc.VectorSubcoreMesh(core_axis_name="c", subcore_axis_name="s")

    @functools.partial(
        pl.kernel,
        mesh=mesh,
        out_type=jax.ShapeDtypeStruct((bsz, seq_len, d), table.dtype),
        scratch_types=(
            [pltpu.VMEM((C, d), table.dtype) for _ in range(NBUF)]
            + [pltpu.SemaphoreType.DMA]
            + [pltpu.SemaphoreType.DMA for _ in range(NBUF)]
        ),
    )
    def sc_copy(table_hbm, out_hbm, *refs):
        bufs = refs[:NBUF]
        gsem = refs[NBUF]
        wsems = refs[NBUF + 1 :]
        wid = lax.axis_index("s") * NC + lax.axis_index("c")
        base = wid * rows_per_w
        gathers = [None] * NBUF
        scatters = [[] for _ in range(NBUF)]
        for j in range(min(AHEAD, n_chunks)):
            gathers[j % NBUF] = pltpu.async_copy(
                table_hbm.at[pl.ds(base + j * C, C)], bufs[j % NBUF], gsem
            )
        for i in range(n_chunks):
            k = i % NBUF
            gathers[k].wait()
            start = base + i * C
            for b in range(bsz):
                scatters[k].append(
                    pltpu.async_copy(bufs[k], out_hbm.at[b, pl.ds(start, C)], wsems[k])
                )
            nxt = i + AHEAD
            if nxt < n_chunks:
                nk = nxt % NBUF
                for h in scatters[nk]:
                    h.wait()
                scatters[nk] = []
                gathers[nk] = pltpu.async_copy(
                    table_hbm.at[pl.ds(base + nxt * C, C)], bufs[nk], gsem
                )
        for k in range(NBUF):
            for h in scatters[k]:
                h.wait()

    return sc_copy(table)


# speedup vs baseline: 2.1728x; 1.0596x over previous
import functools
import jax
import jax.numpy as jnp
from jax import lax
from jax.experimental import pallas as pl
from jax.experimental.pallas import tpu as pltpu
from jax.experimental.pallas import tpu_sc as plsc


def kernel(x_ids, table):
    bsz, seq_len = x_ids.shape
    d = table.shape[1]

    info = plsc.get_sparse_core_info()
    NC, NS = info.num_cores, info.num_subcores
    NW = NC * NS
    rows_per_w = seq_len // NW
    C = 16
    NBUF = 4
    AHEAD = 2
    n_chunks = rows_per_w // C

    mesh = plsc.VectorSubcoreMesh(core_axis_name="c", subcore_axis_name="s")

    @functools.partial(
        pl.kernel,
        mesh=mesh,
        out_type=jax.ShapeDtypeStruct((bsz, seq_len, d), table.dtype),
        scratch_types=(
            [pltpu.VMEM((C, d), table.dtype) for _ in range(NBUF)]
            + [pltpu.SemaphoreType.DMA]
            + [pltpu.SemaphoreType.DMA for _ in range(NBUF)]
        ),
    )
    def sc_copy(table_hbm, out_hbm, *refs):
        bufs = refs[:NBUF]
        gsem = refs[NBUF]
        wsems = refs[NBUF + 1 :]
        wid = lax.axis_index("s") * NC + lax.axis_index("c")
        base = wid * rows_per_w
        gathers = [None] * NBUF
        scatters = [[] for _ in range(NBUF)]
        for j in range(min(AHEAD, n_chunks)):
            gathers[j % NBUF] = pltpu.async_copy(
                table_hbm.at[pl.ds(base + j * C, C)], bufs[j % NBUF], gsem
            )
        for i in range(n_chunks):
            k = i % NBUF
            gathers[k].wait()
            nxt = i + AHEAD
            if nxt < n_chunks:
                nk = nxt % NBUF
                for h in scatters[nk]:
                    h.wait()
                scatters[nk] = []
                gathers[nk] = pltpu.async_copy(
                    table_hbm.at[pl.ds(base + nxt * C, C)], bufs[nk], gsem
                )
            start = base + i * C
            for bi in range(bsz):
                b = lax.rem(bi + wid, bsz)
                scatters[k].append(
                    pltpu.async_copy(bufs[k], out_hbm.at[b, pl.ds(start, C)], wsems[k])
                )
        for k in range(NBUF):
            for h in scatters[k]:
                h.wait()

    return sc_copy(table)


# speedup vs baseline: 2.2257x; 1.0244x over previous
import functools
import jax
import jax.numpy as jnp
from jax import lax
from jax.experimental import pallas as pl
from jax.experimental.pallas import tpu as pltpu
from jax.experimental.pallas import tpu_sc as plsc


def kernel(x_ids, table):
    bsz, seq_len = x_ids.shape
    d = table.shape[1]

    info = plsc.get_sparse_core_info()
    NC, NS = info.num_cores, info.num_subcores
    NW = NC * NS
    rows_per_w = seq_len // NW
    CMAX = 56
    NBUF = 2
    AHEAD = 1
    sizes = []
    left = rows_per_w
    while left > 0:
        c = min(CMAX, left)
        sizes.append(c)
        left -= c
    offs = [sum(sizes[:i]) for i in range(len(sizes))]
    n_chunks = len(sizes)

    mesh = plsc.VectorSubcoreMesh(core_axis_name="c", subcore_axis_name="s")

    @functools.partial(
        pl.kernel,
        mesh=mesh,
        out_type=jax.ShapeDtypeStruct((bsz, seq_len, d), table.dtype),
        scratch_types=(
            [pltpu.VMEM((CMAX, d), table.dtype) for _ in range(NBUF)]
            + [pltpu.SemaphoreType.DMA]
            + [pltpu.SemaphoreType.DMA for _ in range(NBUF)]
        ),
    )
    def sc_copy(table_hbm, out_hbm, *refs):
        bufs = refs[:NBUF]
        gsem = refs[NBUF]
        wsems = refs[NBUF + 1 :]
        wid = lax.axis_index("s") * NC + lax.axis_index("c")
        base = wid * rows_per_w
        gathers = [None] * NBUF
        scatters = [[] for _ in range(NBUF)]

        def gather(j):
            return pltpu.async_copy(
                table_hbm.at[pl.ds(base + offs[j], sizes[j])],
                bufs[j % NBUF].at[pl.ds(0, sizes[j])],
                gsem,
            )

        for j in range(min(AHEAD, n_chunks)):
            gathers[j % NBUF] = gather(j)
        for i in range(n_chunks):
            k = i % NBUF
            gathers[k].wait()
            nxt = i + AHEAD
            if nxt < n_chunks:
                nk = nxt % NBUF
                for h in scatters[nk]:
                    h.wait()
                scatters[nk] = []
                gathers[nk] = gather(nxt)
            start = base + offs[i]
            for b in range(bsz):
                scatters[k].append(
                    pltpu.async_copy(
                        bufs[k].at[pl.ds(0, sizes[i])],
                        out_hbm.at[b, pl.ds(start, sizes[i])],
                        wsems[k],
                    )
                )
        for k in range(NBUF):
            for h in scatters[k]:
                h.wait()

    return sc_copy(table)


# speedup vs baseline: 2.2362x; 1.0047x over previous
import functools
import jax
import jax.numpy as jnp
from jax import lax
from jax.experimental import pallas as pl
from jax.experimental.pallas import tpu as pltpu
from jax.experimental.pallas import tpu_sc as plsc


def kernel(x_ids, table):
    bsz, seq_len = x_ids.shape
    d = table.shape[1]

    info = plsc.get_sparse_core_info()
    NC, NS = info.num_cores, info.num_subcores
    NW = NC * NS
    rows_per_w = seq_len // NW
    NBUF = 2
    AHEAD = 1
    buf_rows = (64, 56)
    sizes = []
    left = rows_per_w
    while left > 0:
        c = min(buf_rows[len(sizes) % NBUF], left)
        sizes.append(c)
        left -= c
    assert all(c % 8 == 0 for c in sizes)
    offs = [sum(sizes[:i]) for i in range(len(sizes))]
    n_chunks = len(sizes)

    mesh = plsc.VectorSubcoreMesh(core_axis_name="c", subcore_axis_name="s")

    @functools.partial(
        pl.kernel,
        mesh=mesh,
        out_type=jax.ShapeDtypeStruct((bsz, seq_len, d), table.dtype),
        scratch_types=(
            [pltpu.VMEM((r, d), table.dtype) for r in buf_rows]
            + [pltpu.SemaphoreType.DMA]
            + [pltpu.SemaphoreType.DMA for _ in range(NBUF)]
        ),
    )
    def sc_copy(table_hbm, out_hbm, *refs):
        bufs = refs[:NBUF]
        gsem = refs[NBUF]
        wsems = refs[NBUF + 1 :]
        wid = lax.axis_index("s") * NC + lax.axis_index("c")
        base = wid * rows_per_w
        gathers = [None] * NBUF
        scatters = [[] for _ in range(NBUF)]

        def gather(j):
            return pltpu.async_copy(
                table_hbm.at[pl.ds(base + offs[j], sizes[j])],
                bufs[j % NBUF].at[pl.ds(0, sizes[j])],
                gsem,
            )

        for j in range(min(AHEAD, n_chunks)):
            gathers[j % NBUF] = gather(j)
        for i in range(n_chunks):
            k = i % NBUF
            gathers[k].wait()
            nxt = i + AHEAD
            if nxt < n_chunks:
                nk = nxt % NBUF
                for h in scatters[nk]:
                    h.wait()
                scatters[nk] = []
                gathers[nk] = gather(nxt)
            start = base + offs[i]
            for b in range(bsz):
                scatters[k].append(
                    pltpu.async_copy(
                        bufs[k].at[pl.ds(0, sizes[i])],
                        out_hbm.at[b, pl.ds(start, sizes[i])],
                        wsems[k],
                    )
                )
        for k in range(NBUF):
            for h in scatters[k]:
                h.wait()

    return sc_copy(table)
